# Initial kernel scaffold; baseline (speedup 1.0000x reference)
import jax
import jax.numpy as jnp
from jax.experimental import pallas as pl


def kernel(x, index, reference):
    raise NotImplementedError("write your pallas kernel here")


# speedup vs baseline: 3.4926x; 3.4926x over previous
import functools

import jax
import jax.numpy as jnp
from jax import lax
from jax.experimental import pallas as pl
from jax.experimental.pallas import tpu as pltpu
from jax.experimental.pallas import tpu_sc as plsc


def _scatter_add_sc(x, index, n_rows):
    E, D = x.shape
    info = plsc.get_sparse_core_info()
    NC, NS, L = info.num_cores, info.num_subcores, info.num_lanes
    DH = D // NC
    B = 128
    NCH = E // B
    RPT = n_rows // NS
    ZR = 125

    mesh = plsc.VectorSubcoreMesh(core_axis_name="c", subcore_axis_name="s")

    @functools.partial(
        pl.kernel,
        out_type=jax.ShapeDtypeStruct((n_rows, D), jnp.float32),
        mesh=mesh,
        scratch_types=[
            pltpu.VMEM_SHARED((n_rows, DH), jnp.float32),
            pltpu.VMEM((2, B), jnp.int32),
            pltpu.VMEM((2, B, DH), jnp.float32),
            pltpu.VMEM((ZR, DH), jnp.float32),
        ],
        compiler_params=pltpu.CompilerParams(use_tc_tiling_on_sc=False),
    )
    def k(x_hbm, idx_hbm, out_hbm, acc, idx_v, upd_v, zbuf):
        c = lax.axis_index("c")
        s = lax.axis_index("s")
        col0 = c * DH

        zero = jnp.zeros((L,), jnp.float32)
        qn = DH // L

        def zstore(i, carry):
            r = i // qn
            q = i - r * qn
            zbuf[r, pl.ds(q * L, L)] = zero
            return carry

        lax.fori_loop(0, ZR * qn, zstore, 0)
        r0 = s * RPT
        for kk in range(RPT // ZR):
            pltpu.sync_copy(zbuf, acc.at[pl.ds(r0 + kk * ZR, ZR)])
        plsc.subcore_barrier()

        lo = (NCH * s) // NS
        hi = (NCH * (s + 1)) // NS

        def chunk(j, carry):
            base = j * B
            pltpu.sync_copy(idx_hbm.at[pl.ds(base, B)], idx_v.at[0])
            pltpu.sync_copy(x_hbm.at[pl.ds(base, B), pl.ds(col0, DH)],
                            upd_v.at[0])
            pltpu.sync_copy(upd_v.at[0], acc.at[idx_v.at[0]], add=True)
            return carry

        lax.fori_loop(lo, hi, chunk, 0)
        plsc.subcore_barrier()

        pltpu.sync_copy(acc.at[pl.ds(r0, RPT)],
                        out_hbm.at[pl.ds(r0, RPT), pl.ds(col0, DH)])

    return k(x, index)


def kernel(x, index, reference):
    return _scatter_add_sc(x, index, reference.shape[0])


# speedup vs baseline: 9.9327x; 2.8439x over previous
import functools

import jax
import jax.numpy as jnp
from jax import lax
from jax.experimental import pallas as pl
from jax.experimental.pallas import tpu as pltpu
from jax.experimental.pallas import tpu_sc as plsc


def _scatter_add_sc(x, index, n_rows):
    E, D = x.shape
    info = plsc.get_sparse_core_info()
    NC, NS, L = info.num_cores, info.num_subcores, info.num_lanes
    DH = D // NC
    B = 80
    EPT = E // NS
    CH = EPT // B
    NBUF = 5
    NGRP = CH // NBUF
    RPT = n_rows // NS
    ZR = 125

    mesh = plsc.VectorSubcoreMesh(core_axis_name="c", subcore_axis_name="s")

    @functools.partial(
        pl.kernel,
        out_type=jax.ShapeDtypeStruct((n_rows, D), jnp.float32),
        mesh=mesh,
        scratch_types=[
            pltpu.VMEM_SHARED((n_rows, DH), jnp.float32),
            pltpu.VMEM((CH, B), jnp.int32),
            pltpu.VMEM((NBUF, B, DH), jnp.float32),
            pltpu.VMEM((ZR, DH), jnp.float32),
            pltpu.SemaphoreType.DMA((NBUF,)),
        ],
        compiler_params=pltpu.CompilerParams(use_tc_tiling_on_sc=False),
    )
    def k(x_hbm, idx_hbm, out_hbm, acc, idx_v, xbuf, zbuf, xsem):
        c = lax.axis_index("c")
        s = lax.axis_index("s")
        col0 = c * DH
        e0 = s * EPT

        def x_copy(b, t):
            base = e0 + t * B
            return pltpu.make_async_copy(
                x_hbm.at[pl.ds(base, B), pl.ds(col0, DH)],
                xbuf.at[b], xsem.at[b])

        def start_load(b, t):
            base = e0 + t * B
            pltpu.async_copy(
                x_hbm.at[pl.ds(base, B), pl.ds(col0, DH)],
                xbuf.at[b], xsem.at[b])

        for b in range(NBUF):
            start_load(b, b)
        pltpu.sync_copy(idx_hbm.at[s], idx_v)

        zero = jnp.zeros((L,), jnp.float32)
        qn = DH // L

        def zstore(i, carry):
            r = i // qn
            q = i - r * qn
            zbuf[r, pl.ds(q * L, L)] = zero
            return carry

        lax.fori_loop(0, ZR * qn, zstore, 0)
        r0 = s * RPT
        for kk in range(RPT // ZR):
            pltpu.sync_copy(zbuf, acc.at[pl.ds(r0 + kk * ZR, ZR)])
        plsc.subcore_barrier()

        def grp(g, carry):
            for b in range(NBUF):
                t = g * NBUF + b
                x_copy(b, t).wait()
                pltpu.sync_copy(xbuf.at[b], acc.at[idx_v.at[t]], add=True)

                @pl.when(g < NGRP - 1)
                def _():
                    start_load(b, t + NBUF)
            return carry

        lax.fori_loop(0, NGRP, grp, 0)
        plsc.subcore_barrier()

        pltpu.sync_copy(acc.at[pl.ds(r0, RPT)],
                        out_hbm.at[pl.ds(r0, RPT), pl.ds(col0, DH)])

    return k(x, index.reshape(NS, CH, B))


def kernel(x, index, reference):
    return _scatter_add_sc(x, index, reference.shape[0])


# speedup vs baseline: 9.9340x; 1.0001x over previous
import functools

import jax
import jax.numpy as jnp
from jax import lax
from jax.experimental import pallas as pl
from jax.experimental.pallas import tpu as pltpu
from jax.experimental.pallas import tpu_sc as plsc


def _scatter_add_sc(x, index, n_rows):
    E, D = x.shape
    info = plsc.get_sparse_core_info()
    NC, NS, L = info.num_cores, info.num_subcores, info.num_lanes
    DH = D // NC
    B = 80
    EPT = E // NS
    CH = EPT // B
    NBUF = 5
    NGRP = CH // NBUF
    RPT = n_rows // NS
    ZR = 125

    mesh = plsc.VectorSubcoreMesh(core_axis_name="c", subcore_axis_name="s")

    @functools.partial(
        pl.kernel,
        out_type=jax.ShapeDtypeStruct((n_rows, D), jnp.float32),
        mesh=mesh,
        scratch_types=[
            pltpu.VMEM_SHARED((n_rows, DH), jnp.float32),
            pltpu.VMEM((CH, B), jnp.int32),
            pltpu.VMEM((NBUF, B, DH), jnp.float32),
            pltpu.VMEM((ZR, DH), jnp.float32),
            pltpu.SemaphoreType.DMA((NBUF,)),
            pltpu.SemaphoreType.DMA((NBUF,)),
        ],
        compiler_params=pltpu.CompilerParams(use_tc_tiling_on_sc=False),
    )
    def k(x_hbm, idx_hbm, out_hbm, acc, idx_v, xbuf, zbuf, xsem, ssem):
        c = lax.axis_index("c")
        s = lax.axis_index("s")
        col0 = c * DH
        e0 = s * EPT

        def x_copy(b, t):
            base = e0 + t * B
            return pltpu.make_async_copy(
                x_hbm.at[pl.ds(base, B), pl.ds(col0, DH)],
                xbuf.at[b], xsem.at[b])

        def start_load(b, t):
            base = e0 + t * B
            pltpu.async_copy(
                x_hbm.at[pl.ds(base, B), pl.ds(col0, DH)],
                xbuf.at[b], xsem.at[b])

        for b in range(NBUF):
            start_load(b, b)
        pltpu.sync_copy(idx_hbm.at[s], idx_v)

        zero = jnp.zeros((L,), jnp.float32)
        qn = DH // L

        def zstore(r, carry):
            for q in range(qn):
                zbuf[r, pl.ds(q * L, L)] = zero
            return carry

        lax.fori_loop(0, ZR, zstore, 0)
        r0 = s * RPT
        for kk in range(RPT // ZR):
            pltpu.sync_copy(zbuf, acc.at[pl.ds(r0 + kk * ZR, ZR)])
        plsc.subcore_barrier()

        def sc_copy(b, t):
            return pltpu.make_async_copy(
                xbuf.at[b], acc.at[idx_v.at[t]], ssem.at[b])

        def grp(g, carry):
            for b in range(NBUF):
                t = g * NBUF + b
                x_copy(b, t).wait()
                pltpu.async_copy(xbuf.at[b], acc.at[idx_v.at[t]],
                                 ssem.at[b], add=True)
                bp = (b - 1) % NBUF

                def drain_refill(tp=t - 1, bp=bp):
                    sc_copy(bp, tp).wait()

                    @pl.when(tp + NBUF < CH)
                    def _():
                        start_load(bp, tp + NBUF)

                if b == 0:
                    @pl.when(g > 0)
                    def _():
                        drain_refill()
                else:
                    drain_refill()
            return carry

        lax.fori_loop(0, NGRP, grp, 0)
        sc_copy(NBUF - 1, CH - 1).wait()
        plsc.subcore_barrier()

        pltpu.sync_copy(acc.at[pl.ds(r0, RPT)],
                        out_hbm.at[pl.ds(r0, RPT), pl.ds(col0, DH)])

    return k(x, index.reshape(NS, CH, B))


def kernel(x, index, reference):
    return _scatter_add_sc(x, index, reference.shape[0])
